# baseline (speedup 1.0000x reference)
import functools

import jax
import jax.numpy as jnp
from jax import lax
from jax.experimental import pallas as pl
from jax.experimental.pallas import tpu as pltpu
from jax.experimental.pallas import tpu_sc as plsc

VOCAB = 1000000
EMB = 64
SEQ = 200
BATCH = 4096

NC = 2
NS = 16
NW = NC * NS
B_PER_W = BATCH // NW

CHUNK0 = 128
CHUNK1 = SEQ - CHUNK0


def _gather_start(table_hbm, idx_v, rows_v, sem, j, buf):
    pltpu.make_async_copy(
        table_hbm.at[idx_v.at[j, pl.ds(0, CHUNK0)]],
        rows_v.at[buf, pl.ds(0, CHUNK0), :],
        sem.at[buf],
    ).start()
    pltpu.make_async_copy(
        table_hbm.at[idx_v.at[j, pl.ds(CHUNK0, CHUNK1)]],
        rows_v.at[buf, pl.ds(CHUNK0, CHUNK1), :],
        sem.at[buf],
    ).start()


def _gather_wait(table_hbm, idx_v, rows_v, sem, j, buf):
    pltpu.make_async_copy(
        table_hbm.at[idx_v.at[j, pl.ds(0, CHUNK0)]],
        rows_v.at[buf, pl.ds(0, CHUNK0), :],
        sem.at[buf],
    ).wait()
    pltpu.make_async_copy(
        table_hbm.at[idx_v.at[j, pl.ds(CHUNK0, CHUNK1)]],
        rows_v.at[buf, pl.ds(CHUNK0, CHUNK1), :],
        sem.at[buf],
    ).wait()


def _accumulate(rows_v, acc_v, j, buf):
    def body(l, carry):
        a0, a1, a2, a3 = carry
        a0 = a0 + rows_v[buf, l, pl.ds(0, 16)]
        a1 = a1 + rows_v[buf, l, pl.ds(16, 16)]
        a2 = a2 + rows_v[buf, l, pl.ds(32, 16)]
        a3 = a3 + rows_v[buf, l, pl.ds(48, 16)]
        return (a0, a1, a2, a3)

    z = jnp.zeros((16,), jnp.float32)
    a0, a1, a2, a3 = lax.fori_loop(0, SEQ, body, (z, z, z, z), unroll=4)
    acc_v[j, pl.ds(0, 16)] = a0
    acc_v[j, pl.ds(16, 16)] = a1
    acc_v[j, pl.ds(32, 16)] = a2
    acc_v[j, pl.ds(48, 16)] = a3


def _sc_body(xt_hbm, table_hbm, out_hbm, idx_v, rows_v, acc_v, sem):
    wid = lax.axis_index("s") * NC + lax.axis_index("c")
    base = wid * B_PER_W

    pltpu.sync_copy(xt_hbm.at[pl.ds(base, B_PER_W), :], idx_v)

    _gather_start(table_hbm, idx_v, rows_v, sem, 0, 0)
    _gather_start(table_hbm, idx_v, rows_v, sem, 1, 1)

    def step(j0, _):
        _gather_wait(table_hbm, idx_v, rows_v, sem, j0, 0)
        _gather_start(table_hbm, idx_v, rows_v, sem, j0 + 2, 0)
        _accumulate(rows_v, acc_v, j0, 0)
        _gather_wait(table_hbm, idx_v, rows_v, sem, j0 + 1, 1)
        _gather_start(table_hbm, idx_v, rows_v, sem, j0 + 3, 1)
        _accumulate(rows_v, acc_v, j0 + 1, 1)
        return 0

    lax.fori_loop(0, (B_PER_W - 2) // 2, lambda i, c: step(2 * i, c), 0,
                  unroll=1)

    _gather_wait(table_hbm, idx_v, rows_v, sem, B_PER_W - 2, 0)
    _accumulate(rows_v, acc_v, B_PER_W - 2, 0)
    _gather_wait(table_hbm, idx_v, rows_v, sem, B_PER_W - 1, 1)
    _accumulate(rows_v, acc_v, B_PER_W - 1, 1)

    pltpu.sync_copy(acc_v, out_hbm.at[pl.ds(base, B_PER_W), :])


@jax.jit
def _sc_sums(xt, table):
    mesh = plsc.VectorSubcoreMesh(core_axis_name="c", subcore_axis_name="s")
    return pl.kernel(
        _sc_body,
        out_type=jax.ShapeDtypeStruct((BATCH, EMB), jnp.float32),
        mesh=mesh,
        compiler_params=pltpu.CompilerParams(use_tc_tiling_on_sc=False),
        scratch_types=[
            pltpu.VMEM((B_PER_W, SEQ), jnp.int32),
            pltpu.VMEM((2, SEQ, EMB), jnp.float32),
            pltpu.VMEM((B_PER_W, EMB), jnp.float32),
            pltpu.SemaphoreType.DMA((2,)),
        ],
    )(xt, table)


def _tc_body(sums_ref, invlen_ref, w_ref, b_ref, out_ref):
    mean = sums_ref[:] * invlen_ref[:]
    out = lax.dot_general(mean, w_ref[:], (((1,), (1,)), ((), ())),
                          preferred_element_type=jnp.float32)
    out_ref[:] = out + b_ref[:]


@jax.jit
def _tc_linear(sums, lengths, W, b):
    invlen = (1.0 / lengths.astype(jnp.float32))[:, None]
    return pl.pallas_call(
        _tc_body,
        out_shape=jax.ShapeDtypeStruct((BATCH, W.shape[0]), jnp.float32),
    )(sums, invlen, W, b[None, :])


def kernel(x, lengths, table, W, b):
    xt = x.T
    sums = _sc_sums(xt, table)
    return _tc_linear(sums, lengths, W, b)
